# baseline (speedup 1.0000x reference)
import jax
import jax.numpy as jnp
from jax import lax
from jax.experimental import pallas as pl
from jax.experimental.pallas import tpu as pltpu
from jax.experimental.pallas import tpu_sc as plsc

VOCAB = 1000000
HIDDEN = 64
NUM_CHANNEL = 4
STEPS = 200
BATCH = 1024

NW = 32
BATCH_PER_W = BATCH // NW
ROWS = NUM_CHANNEL * STEPS
GBLK = 80
NG = ROWS // GBLK
SUB = 40
NSUB = STEPS // SUB
L = 16
HV = HIDDEN // L


def _rsqrt(v16):
    bits = lax.bitcast_convert_type(v16, jnp.int32)
    y = lax.bitcast_convert_type(
        jnp.int32(0x5F3759DF) - lax.shift_right_logical(bits, 1), jnp.float32)
    for _ in range(3):
        y = y * (1.5 - 0.5 * v16 * y * y)
    return y


def _body(x_hbm, chw_hbm, gamma_hbm, beta_hbm, pos_hbm, table_hbm,
          out_hbm, idx_v, rows_v, out_v, pos_v, gb_v, w_v, sem):
    wid = lax.axis_index("s") * 2 + lax.axis_index("c")

    pltpu.sync_copy(pos_hbm, pos_v)
    pltpu.sync_copy(gamma_hbm, gb_v.at[0])
    pltpu.sync_copy(beta_hbm, gb_v.at[1])
    pltpu.sync_copy(chw_hbm, w_v)

    e = jnp.exp(w_v[...])
    w = e / jnp.sum(e)
    ws = [w[c] for c in range(NUM_CHANNEL)]
    gam = [gb_v[0, pl.ds(k * L, L)] for k in range(HV)]
    bet = [gb_v[1, pl.ds(k * L, L)] for k in range(HV)]

    @pl.loop(0, BATCH_PER_W)
    def batch_loop(bl):
        b = wid * BATCH_PER_W + bl
        pltpu.sync_copy(x_hbm.at[b], idx_v)
        cps = [pltpu.async_copy(table_hbm.at[idx_v.at[k]], rows_v.at[k], sem)
               for k in range(NG)]
        for cp in cps:
            cp.wait()

        for si in range(NSUB):
            kr = [divmod(c * STEPS + si * SUB, GBLK) for c in range(NUM_CHANNEL)]

            @pl.loop(0, SUB, unroll=4)
            def pos_loop(j, _si=si, _kr=kr):
                acc = [ws[0] * rows_v[_kr[0][0], _kr[0][1] + j, pl.ds(k * L, L)]
                       + ws[1] * rows_v[_kr[1][0], _kr[1][1] + j, pl.ds(k * L, L)]
                       + ws[2] * rows_v[_kr[2][0], _kr[2][1] + j, pl.ds(k * L, L)]
                       + ws[3] * rows_v[_kr[3][0], _kr[3][1] + j, pl.ds(k * L, L)]
                       for k in range(HV)]
                tot = (acc[0] + acc[1]) + (acc[2] + acc[3])
                sq = (acc[0] * acc[0] + acc[1] * acc[1]) + \
                     (acc[2] * acc[2] + acc[3] * acc[3])
                mean = jnp.sum(tot) * (1.0 / HIDDEN)
                var = jnp.sum(sq) * (1.0 / HIDDEN) - mean * mean
                rstd = _rsqrt(jnp.full((L,), var + 1e-5, jnp.float32))
                s = _si * SUB + j
                for k in range(HV):
                    out_v[s, pl.ds(k * L, L)] = (
                        (acc[k] - mean) * rstd * gam[k] + bet[k]
                        + pos_v[s, pl.ds(k * L, L)])

        pltpu.sync_copy(out_v, out_hbm.at[b])


@jax.jit
def kernel(x, table, ch_w, ln_gamma, ln_beta, pos_emb):
    x10 = x.reshape(BATCH, NG, GBLK)
    chw16 = jnp.full((L,), -1e30, jnp.float32).at[:NUM_CHANNEL].set(ch_w)

    mesh = plsc.VectorSubcoreMesh(core_axis_name="c", subcore_axis_name="s")
    run = pl.kernel(
        _body,
        out_type=jax.ShapeDtypeStruct((BATCH, STEPS, HIDDEN), jnp.float32),
        mesh=mesh,
        scratch_types=[
            pltpu.VMEM((NG, GBLK), jnp.int32),
            pltpu.VMEM((NG, GBLK, HIDDEN), jnp.float32),
            pltpu.VMEM((STEPS, HIDDEN), jnp.float32),
            pltpu.VMEM((STEPS, HIDDEN), jnp.float32),
            pltpu.VMEM((2, HIDDEN), jnp.float32),
            pltpu.VMEM((L,), jnp.float32),
            pltpu.SemaphoreType.DMA,
        ],
        compiler_params=pltpu.CompilerParams(
            needs_layout_passes=False, use_tc_tiling_on_sc=False),
    )
    return run(x10, chw16, ln_gamma, ln_beta, pos_emb, table)


# speedup vs baseline: 1.0015x; 1.0015x over previous
import jax
import jax.numpy as jnp
from jax import lax
from jax.experimental import pallas as pl
from jax.experimental.pallas import tpu as pltpu
from jax.experimental.pallas import tpu_sc as plsc

VOCAB = 1000000
HIDDEN = 64
NUM_CHANNEL = 4
STEPS = 200
BATCH = 1024

NW = 32
BATCH_PER_W = BATCH // NW
ROWS = NUM_CHANNEL * STEPS
GBLK = 80
NG = ROWS // GBLK
SUB = 40
NSUB = STEPS // SUB
L = 16
HV = HIDDEN // L


def _rsqrt(v16):
    bits = lax.bitcast_convert_type(v16, jnp.int32)
    y = lax.bitcast_convert_type(
        jnp.int32(0x5F3759DF) - lax.shift_right_logical(bits, 1), jnp.float32)
    for _ in range(3):
        y = y * (1.5 - 0.5 * v16 * y * y)
    return y


def _body(x_hbm, chw_hbm, gamma_hbm, beta_hbm, pos_hbm, table_hbm,
          out_hbm, idx_v, rows_v, out_v, pos_v, gb_v, w_v, sem):
    wid = lax.axis_index("s") * 2 + lax.axis_index("c")

    pltpu.sync_copy(pos_hbm, pos_v)
    pltpu.sync_copy(gamma_hbm, gb_v.at[0])
    pltpu.sync_copy(beta_hbm, gb_v.at[1])
    pltpu.sync_copy(chw_hbm, w_v)

    e = jnp.exp(w_v[...])
    w = e / jnp.sum(e)
    ws = [w[c] for c in range(NUM_CHANNEL)]
    gam = [gb_v[0, pl.ds(k * L, L)] for k in range(HV)]
    bet = [gb_v[1, pl.ds(k * L, L)] for k in range(HV)]

    @pl.loop(0, BATCH_PER_W)
    def batch_loop(bl):
        b = wid * BATCH_PER_W + bl
        pltpu.sync_copy(x_hbm.at[b], idx_v)
        cps = [pltpu.async_copy(
                   table_hbm.at[idx_v.at[c, pl.ds(si * SUB, SUB)]],
                   rows_v.at[c * NSUB + si], sem)
               for c in range(NUM_CHANNEL) for si in range(NSUB)]
        for cp in cps:
            cp.wait()

        for si in range(NSUB):
            @pl.loop(0, SUB, unroll=4)
            def pos_loop(j, _si=si):
                acc = [ws[0] * rows_v[_si, j, pl.ds(k * L, L)]
                       + ws[1] * rows_v[NSUB + _si, j, pl.ds(k * L, L)]
                       + ws[2] * rows_v[2 * NSUB + _si, j, pl.ds(k * L, L)]
                       + ws[3] * rows_v[3 * NSUB + _si, j, pl.ds(k * L, L)]
                       for k in range(HV)]
                tot = (acc[0] + acc[1]) + (acc[2] + acc[3])
                sq = (acc[0] * acc[0] + acc[1] * acc[1]) + \
                     (acc[2] * acc[2] + acc[3] * acc[3])
                mean = jnp.sum(tot) * (1.0 / HIDDEN)
                var = jnp.sum(sq) * (1.0 / HIDDEN) - mean * mean
                rstd = _rsqrt(jnp.full((L,), var + 1e-5, jnp.float32))
                s = _si * SUB + j
                for k in range(HV):
                    out_v[s, pl.ds(k * L, L)] = (
                        (acc[k] - mean) * rstd * gam[k] + bet[k]
                        + pos_v[s, pl.ds(k * L, L)])

        pltpu.sync_copy(out_v, out_hbm.at[b])


@jax.jit
def kernel(x, table, ch_w, ln_gamma, ln_beta, pos_emb):
    chw16 = jnp.full((L,), -1e30, jnp.float32).at[:NUM_CHANNEL].set(ch_w)

    mesh = plsc.VectorSubcoreMesh(core_axis_name="c", subcore_axis_name="s")
    run = pl.kernel(
        _body,
        out_type=jax.ShapeDtypeStruct((BATCH, STEPS, HIDDEN), jnp.float32),
        mesh=mesh,
        scratch_types=[
            pltpu.VMEM((NUM_CHANNEL, STEPS), jnp.int32),
            pltpu.VMEM((NUM_CHANNEL * NSUB, SUB, HIDDEN), jnp.float32),
            pltpu.VMEM((STEPS, HIDDEN), jnp.float32),
            pltpu.VMEM((STEPS, HIDDEN), jnp.float32),
            pltpu.VMEM((2, HIDDEN), jnp.float32),
            pltpu.VMEM((L,), jnp.float32),
            pltpu.SemaphoreType.DMA,
        ],
        compiler_params=pltpu.CompilerParams(
            needs_layout_passes=False, use_tc_tiling_on_sc=False),
    )
    return run(x, chw16, ln_gamma, ln_beta, pos_emb, table)
